# baseline (speedup 1.0000x reference)
import functools

import jax
import jax.numpy as jnp
from jax import lax
from jax.experimental import pallas as pl
from jax.experimental.pallas import tpu as pltpu
from jax.experimental.pallas import tpu_sc as plsc

BIG = 1e30
BIG2 = 2e30
ROWS = 512
QB = 512
EPS = 1e-5

_interpret = False
_diag_ref_knn = False
_diag_jnp_gather = False


def _knn_body(b_smem, xa_ref, xb_ref, ba_ref, bb_ref, idx_ref,
              r_scr, base_scr, *, k, n, cw):
    nchunks = n // cw
    i = pl.program_id(0)
    q0 = i * QB
    qlo = b_smem[q0]
    qhi = b_smem[q0 + QB - 1]
    xb = xb_ref[...]
    bb = bb_ref[...]

    def compute_chunk(c, off):
        xa_c = xa_ref[pl.ds(c * cw, cw), :]
        ba_c = ba_ref[pl.ds(c * cw, cw), :]
        sqa = jnp.sum(xa_c * xa_c, axis=1, keepdims=True)
        dots = lax.dot_general(
            xa_c.astype(jnp.bfloat16), xb.astype(jnp.bfloat16),
            (((1,), (1,)), ((), ())),
            preferred_element_type=jnp.float32)
        r = sqa - 2.0 * dots
        r = jnp.where(ba_c != bb, BIG, r)
        r_scr[pl.ds(off * cw, cw), :] = r
        base_scr[off] = c * cw

    compute_chunk(0, 0)
    off = jnp.int32(1)
    for c in range(1, nchunks):
        clo = b_smem[c * cw]
        chi = b_smem[c * cw + cw - 1]
        act = jnp.logical_and(clo <= qhi, chi >= qlo)

        @pl.when(act)
        def _(c=c, off=off):
            compute_chunk(c, off)

        off = off + act.astype(jnp.int32)
    n_act = off

    iota_c = lax.broadcasted_iota(jnp.int32, (cw, QB), 0)
    rows = []
    for _ in range(k):
        def pass1(ci, cur):
            blk = r_scr[pl.ds(ci * cw, cw), :]
            return jnp.minimum(cur, jnp.min(blk, axis=0, keepdims=True))
        m = lax.fori_loop(0, n_act, pass1,
                          jnp.full((1, QB), BIG, jnp.float32))

        def pass2(ci, cur):
            blk = r_scr[pl.ds(ci * cw, cw), :]
            gx = iota_c + base_scr[ci]
            cand = jnp.min(jnp.where(blk == m, gx, n), axis=0, keepdims=True)
            return jnp.minimum(cur, cand)
        j = lax.fori_loop(0, n_act, pass2,
                          jnp.full((1, QB), n, jnp.int32))
        rows.append(j)

        def pass3(ci, _):
            blk = r_scr[pl.ds(ci * cw, cw), :]
            gx = iota_c + base_scr[ci]
            r_scr[pl.ds(ci * cw, cw), :] = jnp.where(
                (blk == m) & (gx == j), BIG2, blk)
            return 0
        lax.fori_loop(0, n_act, pass3, 0)
    idx_ref[...] = jnp.concatenate(rows, axis=0)


def _knn(x, batch1d, bcol, brow, k):
    n, d = x.shape
    cw = 512
    grid_spec = pltpu.PrefetchScalarGridSpec(
        num_scalar_prefetch=1,
        grid=(n // QB,),
        in_specs=[
            pl.BlockSpec((n, d), lambda i, b: (0, 0)),
            pl.BlockSpec((QB, d), lambda i, b: (i, 0)),
            pl.BlockSpec((n, 1), lambda i, b: (0, 0)),
            pl.BlockSpec((1, QB), lambda i, b: (0, i)),
        ],
        out_specs=pl.BlockSpec((k, QB), lambda i, b: (0, i)),
        scratch_shapes=[
            pltpu.VMEM((n, QB), jnp.float32),
            pltpu.SMEM((n // cw,), jnp.int32),
        ],
    )
    return pl.pallas_call(
        functools.partial(_knn_body, k=k, n=n, cw=cw),
        grid_spec=grid_spec,
        out_shape=jax.ShapeDtypeStruct((k, n), jnp.int32),
        compiler_params=pltpu.CompilerParams(
            dimension_semantics=("parallel",)),
        interpret=_interpret,
    )(batch1d, x, x, bcol, brow)


def _sc_gather(table, idx):
    _, dcols = table.shape
    (b_total,) = idx.shape
    info = plsc.get_sparse_core_info()
    nw = info.num_cores * info.num_subcores
    b_per_w = b_total // nw
    mesh = plsc.VectorSubcoreMesh(core_axis_name="c", subcore_axis_name="s")

    @functools.partial(
        pl.kernel, mesh=mesh,
        out_type=jax.ShapeDtypeStruct((b_total, dcols), jnp.float32),
        compiler_params=pltpu.CompilerParams(use_tc_tiling_on_sc=False),
        scratch_types=[
            pltpu.VMEM((b_per_w,), jnp.int32),
            pltpu.VMEM((b_per_w, dcols), jnp.float32),
            pltpu.SemaphoreType.DMA,
        ],
    )
    def gk(table_hbm, idx_hbm, out_hbm, idx_v, rows_v, sem):
        wid = lax.axis_index("s") * info.num_cores + lax.axis_index("c")
        base = wid * b_per_w
        pltpu.sync_copy(idx_hbm.at[pl.ds(base, b_per_w)], idx_v)
        pltpu.async_copy(table_hbm.at[idx_v], rows_v, sem).wait()
        pltpu.sync_copy(rows_v, out_hbm.at[pl.ds(base, b_per_w)])

    return gk(table, idx)


def _accum_stats(s_ref, y):
    ps = jnp.sum(y, axis=0, keepdims=True)
    pss = jnp.sum(y * y, axis=0, keepdims=True)
    s_ref[...] = jnp.concatenate([ps, pss], axis=0)[None]


def _edge_l1_body(x_ref, g_ref, w_ref, b_ref, z_ref, s_ref, *, k):
    xi = x_ref[...]
    xj = g_ref[...]
    r_rows, _, d = xj.shape
    feat = jnp.concatenate(
        [jnp.broadcast_to(xi[:, None, :], xj.shape), xj - xi[:, None, :]],
        axis=2).reshape(r_rows * k, 2 * d)
    z = jnp.maximum(
        jnp.dot(feat.astype(jnp.bfloat16), w_ref[...].astype(jnp.bfloat16),
                preferred_element_type=jnp.float32) + b_ref[...], 0.0)
    d1 = z.shape[1]
    z_ref[...] = z.reshape(r_rows, k, d1)
    _accum_stats(s_ref, z)


def _edge_l1(x, g3, w1, b1, k):
    n, _, d = g3.shape
    d1 = w1.shape[1]
    return pl.pallas_call(
        functools.partial(_edge_l1_body, k=k),
        grid=(n // ROWS,),
        in_specs=[
            pl.BlockSpec((ROWS, d), lambda i: (i, 0)),
            pl.BlockSpec((ROWS, k, d), lambda i: (i, 0, 0)),
            pl.BlockSpec(w1.shape, lambda i: (0, 0)),
            pl.BlockSpec((1, d1), lambda i: (0, 0)),
        ],
        out_specs=[
            pl.BlockSpec((ROWS, k, d1), lambda i: (i, 0, 0)),
            pl.BlockSpec((1, 2, d1), lambda i: (i, 0, 0)),
        ],
        out_shape=[jax.ShapeDtypeStruct((n, k, d1), jnp.float32),
                   jax.ShapeDtypeStruct((n // ROWS, 2, d1), jnp.float32)],
        compiler_params=pltpu.CompilerParams(
            dimension_semantics=("parallel",)),
        interpret=_interpret,
    )(x, g3, w1, b1)


def _norm(z2d, s_ref, g_ref, be_ref, nk):
    s = jnp.sum(s_ref[...], axis=0)
    m = s[0:1] / nk
    v = s[1:2] / nk - m * m
    return g_ref[...] * (z2d - m) / jnp.sqrt(v + EPS) + be_ref[...]


def _edge_next_body(z_ref, s_ref, g_ref, be_ref, w_ref, b_ref,
                    zo_ref, so_ref, *, k, nk):
    z = z_ref[...]
    r_rows, _, d1 = z.shape
    x = _norm(z.reshape(r_rows * k, d1), s_ref, g_ref, be_ref, nk)
    y = jnp.maximum(
        jnp.dot(x.astype(jnp.bfloat16), w_ref[...].astype(jnp.bfloat16),
                preferred_element_type=jnp.float32)
        + b_ref[...], 0.0)
    d2 = y.shape[1]
    zo_ref[...] = y.reshape(r_rows, k, d2)
    _accum_stats(so_ref, y)


def _edge_next(z, s, g, be, w, b, k):
    n, _, d1 = z.shape
    d2 = w.shape[1]
    nk = float(n * k)
    return pl.pallas_call(
        functools.partial(_edge_next_body, k=k, nk=nk),
        grid=(n // ROWS,),
        in_specs=[
            pl.BlockSpec((ROWS, k, d1), lambda i: (i, 0, 0)),
            pl.BlockSpec(s.shape, lambda i: (0, 0, 0)),
            pl.BlockSpec((1, d1), lambda i: (0, 0)),
            pl.BlockSpec((1, d1), lambda i: (0, 0)),
            pl.BlockSpec((d1, d2), lambda i: (0, 0)),
            pl.BlockSpec((1, d2), lambda i: (0, 0)),
        ],
        out_specs=[
            pl.BlockSpec((ROWS, k, d2), lambda i: (i, 0, 0)),
            pl.BlockSpec((1, 2, d2), lambda i: (i, 0, 0)),
        ],
        out_shape=[jax.ShapeDtypeStruct((n, k, d2), jnp.float32),
                   jax.ShapeDtypeStruct((n // ROWS, 2, d2), jnp.float32)],
        compiler_params=pltpu.CompilerParams(
            dimension_semantics=("parallel",)),
        interpret=_interpret,
    )(z, s, g, be, w, b)


def _edge_final_body(z_ref, s_ref, g_ref, be_ref, o_ref, *, k, nk):
    z = z_ref[...]
    r_rows, _, dl = z.shape
    x = _norm(z.reshape(r_rows * k, dl), s_ref, g_ref, be_ref,
              nk).reshape(r_rows, k, dl)
    zmax = x[:, 0, :]
    for j in range(1, k):
        zmax = jnp.maximum(zmax, x[:, j, :])
    o_ref[...] = zmax


def _edge_final(z, s, g, be, k):
    n, _, dl = z.shape
    nk = float(n * k)
    return pl.pallas_call(
        functools.partial(_edge_final_body, k=k, nk=nk),
        grid=(n // ROWS,),
        in_specs=[
            pl.BlockSpec((ROWS, k, dl), lambda i: (i, 0, 0)),
            pl.BlockSpec(s.shape, lambda i: (0, 0, 0)),
            pl.BlockSpec((1, dl), lambda i: (0, 0)),
            pl.BlockSpec((1, dl), lambda i: (0, 0)),
        ],
        out_specs=pl.BlockSpec((ROWS, dl), lambda i: (i, 0)),
        out_shape=jax.ShapeDtypeStruct((n, dl), jnp.float32),
        compiler_params=pltpu.CompilerParams(
            dimension_semantics=("parallel",)),
        interpret=_interpret,
    )(z, s, g, be)


def _dyn_edge_conv(x, batch1d, bcol, brow, params, k, last_raw=False):
    n, _ = x.shape
    if _diag_ref_knn:
        sq = jnp.sum(x * x, axis=1)
        dist = sq[:, None] + sq[None, :] - 2.0 * (x @ x.T)
        dist = jnp.where(bcol != brow, jnp.inf, dist)
        _, idx_d = lax.top_k(-dist, k)
        idx_t = idx_d.T
    else:
        idx_t = _knn(x, batch1d, bcol, brow, k)
    w1, b1, g1, be1 = params[0]
    d = x.shape[1]
    dpad = max(16, -(-d // 16) * 16)
    if dpad != d:
        xg = jnp.pad(x, ((0, 0), (0, dpad - d)))
        w1 = jnp.concatenate([
            jnp.pad(w1[:d], ((0, dpad - d), (0, 0))),
            jnp.pad(w1[d:], ((0, dpad - d), (0, 0)))], axis=0)
    else:
        xg = x
    if _diag_jnp_gather:
        gathered = xg[idx_t.T.reshape(n * k)]
    else:
        gathered = _sc_gather(xg, idx_t.T.reshape(n * k))
    z, s = _edge_l1(xg, gathered.reshape(n, k, dpad), w1,
                    b1.reshape(1, -1), k)
    gp, bep = g1.reshape(1, -1), be1.reshape(1, -1)
    for (w, b, g, be) in params[1:]:
        z, s = _edge_next(z, s, gp, bep, w, b.reshape(1, -1), k)
        gp, bep = g.reshape(1, -1), be.reshape(1, -1)
    if last_raw:
        return z, s, gp, bep
    return _edge_final(z, s, gp, bep, k)


def _head_body(z_ref, s_ref, gl_ref, bel_ref, bc_ref,
               w1_ref, b1_ref, g1_ref, be1_ref,
               w2_ref, b2_ref, g2_ref, be2_ref, cw_ref, cb_ref, o_ref,
               *, nseg, kk, nk):
    s = jnp.sum(s_ref[...], axis=0)
    m = s[0:1] / nk
    v = s[1:2] / nk - m * m
    gl = gl_ref[...]
    bel = bel_ref[...]
    x = None
    for j in range(kk):
        zp = z_ref[:, j, :]
        xp = gl * (zp - m) / jnp.sqrt(v + EPS) + bel
        x = xp if x is None else jnp.maximum(x, xp)
    for (w_ref, b_ref, g_ref, be_ref) in ((w1_ref, b1_ref, g1_ref, be1_ref),
                                          (w2_ref, b2_ref, g2_ref, be2_ref)):
        z = jnp.maximum(
            jnp.dot(x.astype(jnp.bfloat16), w_ref[...].astype(jnp.bfloat16),
                    preferred_element_type=jnp.float32)
            + b_ref[...], 0.0)
        mz = jnp.mean(z, axis=0, keepdims=True)
        vz = jnp.mean((z - mz) * (z - mz), axis=0, keepdims=True)
        x = g_ref[...] * (z - mz) / jnp.sqrt(vz + EPS) + be_ref[...]
    bc = bc_ref[...]
    segs = []
    for sid in range(nseg):
        msk = bc == sid
        segs.append(jnp.max(jnp.where(msk, x, -jnp.inf), axis=0,
                            keepdims=True))
    gpool = jnp.concatenate(segs, axis=0)
    logits = (jnp.dot(gpool.astype(jnp.bfloat16),
                      cw_ref[...].astype(jnp.bfloat16),
                      preferred_element_type=jnp.float32)
              + cb_ref[...])
    mx = jnp.max(logits, axis=1, keepdims=True)
    shifted = logits - mx
    lse = jnp.log(jnp.sum(jnp.exp(shifted), axis=1, keepdims=True))
    o_ref[...] = shifted - lse


def _head(z, s, gl, bel, kk, bcol, mlp_params, cls_w, cls_b, nseg):
    (w1, b1, g1, be1), (w2, b2, g2, be2) = mlp_params
    n = z.shape[0]
    ncls = cls_w.shape[1]
    nk = float(n * kk)
    return pl.pallas_call(
        functools.partial(_head_body, nseg=nseg, kk=kk, nk=nk),
        out_shape=jax.ShapeDtypeStruct((nseg, ncls), jnp.float32),
        interpret=_interpret,
    )(z, s, gl.reshape(1, -1), bel.reshape(1, -1), bcol,
      w1, b1.reshape(1, -1), g1.reshape(1, -1), be1.reshape(1, -1),
      w2, b2.reshape(1, -1), g2.reshape(1, -1), be2.reshape(1, -1),
      cls_w, cls_b.reshape(1, -1))


def kernel(pos, batch, conv1_params, conv2_params, mlp_params, cls_W, cls_b):
    n = pos.shape[0]
    batch = batch.astype(jnp.int32)
    bcol = batch.reshape(n, 1)
    brow = batch.reshape(1, n)
    x1 = _dyn_edge_conv(pos, batch, bcol, brow, conv1_params, 20)
    z2, s2, g2, be2 = _dyn_edge_conv(x1, batch, bcol, brow, conv2_params, 10,
                                     last_raw=True)
    return _head(z2, s2, g2, be2, 10, bcol, mlp_params, cls_W, cls_b, 8)


# speedup vs baseline: 1.1843x; 1.1843x over previous
import functools

import jax
import jax.numpy as jnp
from jax import lax
from jax.experimental import pallas as pl
from jax.experimental.pallas import tpu as pltpu
from jax.experimental.pallas import tpu_sc as plsc

BIG = 1e30
BIG2 = 2e30
ROWS = 512
QB = 512
EPS = 1e-5

_interpret = False
_diag_ref_knn = False
_diag_jnp_gather = False


def _knn_body(b_smem, xa_ref, xb_ref, ba_ref, bb_ref, idx_ref,
              r_scr, base_scr, *, k, n, cw):
    nchunks = n // cw
    i = pl.program_id(0)
    q0 = i * QB
    qlo = b_smem[q0]
    qhi = b_smem[q0 + QB - 1]
    xb = xb_ref[...]
    bb = bb_ref[...]

    def compute_chunk(c, off):
        xa_c = xa_ref[pl.ds(c * cw, cw), :]
        ba_c = ba_ref[pl.ds(c * cw, cw), :]
        sqa = jnp.sum(xa_c * xa_c, axis=1, keepdims=True)
        dots = lax.dot_general(
            xa_c.astype(jnp.bfloat16), xb.astype(jnp.bfloat16),
            (((1,), (1,)), ((), ())),
            preferred_element_type=jnp.float32)
        r = sqa - 2.0 * dots
        r = jnp.where(ba_c != bb, BIG, r)
        r_scr[pl.ds(off * cw, cw), :] = r
        base_scr[off] = c * cw

    compute_chunk(0, 0)
    off = jnp.int32(1)
    for c in range(1, nchunks):
        clo = b_smem[c * cw]
        chi = b_smem[c * cw + cw - 1]
        act = jnp.logical_and(clo <= qhi, chi >= qlo)

        @pl.when(act)
        def _(c=c, off=off):
            compute_chunk(c, off)

        off = off + act.astype(jnp.int32)
    n_act = off

    iota_c = lax.broadcasted_iota(jnp.int32, (cw, QB), 0)
    rows = []
    for _ in range(k):
        def pass1(ci, cur):
            blk = r_scr[pl.ds(ci * cw, cw), :]
            return jnp.minimum(cur, jnp.min(blk, axis=0, keepdims=True))
        m = lax.fori_loop(0, n_act, pass1,
                          jnp.full((1, QB), BIG, jnp.float32))

        def pass2(ci, cur):
            blk = r_scr[pl.ds(ci * cw, cw), :]
            gx = iota_c + base_scr[ci]
            cand = jnp.min(jnp.where(blk == m, gx, n), axis=0, keepdims=True)
            return jnp.minimum(cur, cand)
        j = lax.fori_loop(0, n_act, pass2,
                          jnp.full((1, QB), n, jnp.int32))
        rows.append(j)

        def pass3(ci, _):
            blk = r_scr[pl.ds(ci * cw, cw), :]
            gx = iota_c + base_scr[ci]
            r_scr[pl.ds(ci * cw, cw), :] = jnp.where(
                (blk == m) & (gx == j), BIG2, blk)
            return 0
        lax.fori_loop(0, n_act, pass3, 0)
    idx_ref[...] = jnp.concatenate(rows, axis=0)


def _knn(x, batch1d, bcol, brow, k):
    n, d = x.shape
    cw = 512
    grid_spec = pltpu.PrefetchScalarGridSpec(
        num_scalar_prefetch=1,
        grid=(n // QB,),
        in_specs=[
            pl.BlockSpec((n, d), lambda i, b: (0, 0)),
            pl.BlockSpec((QB, d), lambda i, b: (i, 0)),
            pl.BlockSpec((n, 1), lambda i, b: (0, 0)),
            pl.BlockSpec((1, QB), lambda i, b: (0, i)),
        ],
        out_specs=pl.BlockSpec((k, QB), lambda i, b: (0, i)),
        scratch_shapes=[
            pltpu.VMEM((n, QB), jnp.float32),
            pltpu.SMEM((n // cw,), jnp.int32),
        ],
    )
    return pl.pallas_call(
        functools.partial(_knn_body, k=k, n=n, cw=cw),
        grid_spec=grid_spec,
        out_shape=jax.ShapeDtypeStruct((k, n), jnp.int32),
        compiler_params=pltpu.CompilerParams(
            dimension_semantics=("parallel",)),
        interpret=_interpret,
    )(batch1d, x, x, bcol, brow)


def _sc_gather(table, idx):
    _, dcols = table.shape
    (b_total,) = idx.shape
    info = plsc.get_sparse_core_info()
    nw = info.num_cores * info.num_subcores
    b_per_w = b_total // nw
    mesh = plsc.VectorSubcoreMesh(core_axis_name="c", subcore_axis_name="s")

    @functools.partial(
        pl.kernel, mesh=mesh,
        out_type=jax.ShapeDtypeStruct((b_total, dcols), jnp.float32),
        compiler_params=pltpu.CompilerParams(use_tc_tiling_on_sc=False),
        scratch_types=[
            pltpu.VMEM((b_per_w,), jnp.int32),
            pltpu.VMEM((b_per_w, dcols), jnp.float32),
            pltpu.SemaphoreType.DMA,
        ],
    )
    def gk(table_hbm, idx_hbm, out_hbm, idx_v, rows_v, sem):
        wid = lax.axis_index("s") * info.num_cores + lax.axis_index("c")
        base = wid * b_per_w
        pltpu.sync_copy(idx_hbm.at[pl.ds(base, b_per_w)], idx_v)
        pltpu.async_copy(table_hbm.at[idx_v], rows_v, sem).wait()
        pltpu.sync_copy(rows_v, out_hbm.at[pl.ds(base, b_per_w)])

    return gk(table, idx)


def _accum_stats(s_ref, y):
    ps = jnp.sum(y, axis=0, keepdims=True)
    pss = jnp.sum(y * y, axis=0, keepdims=True)
    s_ref[...] = jnp.concatenate([ps, pss], axis=0)[None]


def _edge_l1_body(x_ref, g_ref, w_ref, b_ref, z_ref, s_ref, *, k):
    xi = x_ref[...]
    xj = g_ref[...]
    r_rows, _, d = xj.shape
    feat = jnp.concatenate(
        [jnp.broadcast_to(xi[:, None, :], xj.shape), xj - xi[:, None, :]],
        axis=2).reshape(r_rows * k, 2 * d)
    z = jnp.maximum(
        jnp.dot(feat.astype(jnp.bfloat16), w_ref[...].astype(jnp.bfloat16),
                preferred_element_type=jnp.float32) + b_ref[...], 0.0)
    d1 = z.shape[1]
    z_ref[...] = z.reshape(r_rows, k, d1)
    _accum_stats(s_ref, z)


def _edge_l1(x, g3, w1, b1, k):
    n, _, d = g3.shape
    d1 = w1.shape[1]
    return pl.pallas_call(
        functools.partial(_edge_l1_body, k=k),
        grid=(n // ROWS,),
        in_specs=[
            pl.BlockSpec((ROWS, d), lambda i: (i, 0)),
            pl.BlockSpec((ROWS, k, d), lambda i: (i, 0, 0)),
            pl.BlockSpec(w1.shape, lambda i: (0, 0)),
            pl.BlockSpec((1, d1), lambda i: (0, 0)),
        ],
        out_specs=[
            pl.BlockSpec((ROWS, k, d1), lambda i: (i, 0, 0)),
            pl.BlockSpec((1, 2, d1), lambda i: (i, 0, 0)),
        ],
        out_shape=[jax.ShapeDtypeStruct((n, k, d1), jnp.float32),
                   jax.ShapeDtypeStruct((n // ROWS, 2, d1), jnp.float32)],
        compiler_params=pltpu.CompilerParams(
            dimension_semantics=("parallel",)),
        interpret=_interpret,
    )(x, g3, w1, b1)


def _norm(z2d, s_ref, g_ref, be_ref, nk):
    s = jnp.sum(s_ref[...], axis=0)
    m = s[0:1] / nk
    v = s[1:2] / nk - m * m
    return g_ref[...] * (z2d - m) / jnp.sqrt(v + EPS) + be_ref[...]


def _edge_next_body(z_ref, s_ref, g_ref, be_ref, w_ref, b_ref,
                    zo_ref, so_ref, *, k, nk):
    z = z_ref[...]
    r_rows, _, d1 = z.shape
    x = _norm(z.reshape(r_rows * k, d1), s_ref, g_ref, be_ref, nk)
    y = jnp.maximum(
        jnp.dot(x.astype(jnp.bfloat16), w_ref[...].astype(jnp.bfloat16),
                preferred_element_type=jnp.float32)
        + b_ref[...], 0.0)
    d2 = y.shape[1]
    zo_ref[...] = y.reshape(r_rows, k, d2)
    _accum_stats(so_ref, y)


def _edge_next(z, s, g, be, w, b, k):
    n, _, d1 = z.shape
    d2 = w.shape[1]
    nk = float(n * k)
    return pl.pallas_call(
        functools.partial(_edge_next_body, k=k, nk=nk),
        grid=(n // ROWS,),
        in_specs=[
            pl.BlockSpec((ROWS, k, d1), lambda i: (i, 0, 0)),
            pl.BlockSpec(s.shape, lambda i: (0, 0, 0)),
            pl.BlockSpec((1, d1), lambda i: (0, 0)),
            pl.BlockSpec((1, d1), lambda i: (0, 0)),
            pl.BlockSpec((d1, d2), lambda i: (0, 0)),
            pl.BlockSpec((1, d2), lambda i: (0, 0)),
        ],
        out_specs=[
            pl.BlockSpec((ROWS, k, d2), lambda i: (i, 0, 0)),
            pl.BlockSpec((1, 2, d2), lambda i: (i, 0, 0)),
        ],
        out_shape=[jax.ShapeDtypeStruct((n, k, d2), jnp.float32),
                   jax.ShapeDtypeStruct((n // ROWS, 2, d2), jnp.float32)],
        compiler_params=pltpu.CompilerParams(
            dimension_semantics=("parallel",)),
        interpret=_interpret,
    )(z, s, g, be, w, b)


def _edge_final_body(z_ref, s_ref, g_ref, be_ref, o_ref, *, k, nk):
    z = z_ref[...]
    r_rows, _, dl = z.shape
    x = _norm(z.reshape(r_rows * k, dl), s_ref, g_ref, be_ref,
              nk).reshape(r_rows, k, dl)
    zmax = x[:, 0, :]
    for j in range(1, k):
        zmax = jnp.maximum(zmax, x[:, j, :])
    o_ref[...] = zmax


def _edge_final(z, s, g, be, k):
    n, _, dl = z.shape
    nk = float(n * k)
    return pl.pallas_call(
        functools.partial(_edge_final_body, k=k, nk=nk),
        grid=(n // ROWS,),
        in_specs=[
            pl.BlockSpec((ROWS, k, dl), lambda i: (i, 0, 0)),
            pl.BlockSpec(s.shape, lambda i: (0, 0, 0)),
            pl.BlockSpec((1, dl), lambda i: (0, 0)),
            pl.BlockSpec((1, dl), lambda i: (0, 0)),
        ],
        out_specs=pl.BlockSpec((ROWS, dl), lambda i: (i, 0)),
        out_shape=jax.ShapeDtypeStruct((n, dl), jnp.float32),
        compiler_params=pltpu.CompilerParams(
            dimension_semantics=("parallel",)),
        interpret=_interpret,
    )(z, s, g, be)


def _dyn_edge_conv(x, batch1d, bcol, brow, params, k, last_raw=False):
    n, _ = x.shape
    if _diag_ref_knn:
        sq = jnp.sum(x * x, axis=1)
        dist = sq[:, None] + sq[None, :] - 2.0 * (x @ x.T)
        dist = jnp.where(bcol != brow, jnp.inf, dist)
        _, idx_d = lax.top_k(-dist, k)
        idx_t = idx_d.T
    else:
        idx_t = _knn(x, batch1d, bcol, brow, k)
    w1, b1, g1, be1 = params[0]
    d = x.shape[1]
    dpad = max(16, -(-d // 16) * 16)
    if dpad != d:
        xg = jnp.pad(x, ((0, 0), (0, dpad - d)))
        w1 = jnp.concatenate([
            jnp.pad(w1[:d], ((0, dpad - d), (0, 0))),
            jnp.pad(w1[d:], ((0, dpad - d), (0, 0)))], axis=0)
    else:
        xg = x
    if not last_raw:
        if _diag_jnp_gather:
            gathered = xg[idx_t.reshape(k * n)]
        else:
            gathered = _sc_gather(xg, idx_t.reshape(k * n))
        layers = [(w1, b1, g1, be1)] + list(params[1:])
        return _fused_conv(xg, gathered.reshape(k, n, dpad), layers, k)
    if _diag_jnp_gather:
        gathered = xg[idx_t.T.reshape(n * k)]
    else:
        gathered = _sc_gather(xg, idx_t.T.reshape(n * k))
    z, s = _edge_l1(xg, gathered.reshape(n, k, dpad), w1,
                    b1.reshape(1, -1), k)
    gp, bep = g1.reshape(1, -1), be1.reshape(1, -1)
    for (w, b, g, be) in params[1:]:
        z, s = _edge_next(z, s, gp, bep, w, b.reshape(1, -1), k)
        gp, bep = g.reshape(1, -1), be.reshape(1, -1)
    if last_raw:
        return z, s, gp, bep
    return _edge_final(z, s, gp, bep, k)


def _fconv_body(x_ref, g_ref, *refs, k, dpad, dims, nk, n, nlayer):
    lrefs = [refs[4 * l:4 * l + 4] for l in range(nlayer)]
    o_ref = refs[4 * nlayer]
    xt_scr = refs[4 * nlayer + 1]
    z_scrs = refs[4 * nlayer + 2:4 * nlayer + 2 + nlayer]
    j = pl.program_id(0)
    d1 = dims[0]

    @pl.when(j == 0)
    def _():
        xt_scr[...] = jnp.transpose(x_ref[...])

    xt = xt_scr[...]
    xjt = jnp.transpose(g_ref[0])
    featt = jnp.concatenate([xt, xjt - xt], axis=0)
    w1t = jnp.transpose(lrefs[0][0][...])
    z1 = jnp.maximum(
        lax.dot_general(w1t.astype(jnp.bfloat16), featt.astype(jnp.bfloat16),
                        (((1,), (0,)), ((), ())),
                        preferred_element_type=jnp.float32)
        + lrefs[0][1][...], 0.0)
    z_scrs[0][pl.ds(j * d1, d1), :] = z1

    @pl.when(j == k - 1)
    def _():
        zin = z_scrs[0]
        din = d1
        for l in range(1, nlayer + 1):
            s = jnp.zeros((din, 1), jnp.float32)
            ss = jnp.zeros((din, 1), jnp.float32)
            panels = []
            for jj in range(k):
                zp = zin[jj * din:(jj + 1) * din, :]
                panels.append(zp)
                s = s + jnp.sum(zp, axis=1, keepdims=True)
                ss = ss + jnp.sum(zp * zp, axis=1, keepdims=True)
            m = s / nk
            v = ss / nk - m * m
            gcol = jnp.transpose(lrefs[l - 1][2][...])
            becol = jnp.transpose(lrefs[l - 1][3][...])
            inv = jnp.sqrt(v + EPS)
            if l < nlayer:
                dout = dims[l]
                wt = jnp.transpose(lrefs[l][0][...]).astype(jnp.bfloat16)
                bcol = lrefs[l][1][...]
                for jj in range(k):
                    xn = gcol * (panels[jj] - m) / inv + becol
                    zl = jnp.maximum(
                        lax.dot_general(wt, xn.astype(jnp.bfloat16),
                                        (((1,), (0,)), ((), ())),
                                        preferred_element_type=jnp.float32)
                        + bcol, 0.0)
                    z_scrs[l][pl.ds(jj * dout, dout), :] = zl
                zin = z_scrs[l]
                din = dout
            else:
                acc = None
                for jj in range(k):
                    xn = gcol * (panels[jj] - m) / inv + becol
                    acc = xn if acc is None else jnp.maximum(acc, xn)
                o_ref[...] = jnp.transpose(acc)


def _fused_conv(x, gathered_knd, layers, k):
    n, dpad = x.shape
    nlayer = len(layers)
    dims = [w.shape[1] for (w, _, _, _) in layers]
    nk = float(n * k)
    args = [x, gathered_knd]
    in_specs = [
        pl.BlockSpec((n, dpad), lambda j: (0, 0)),
        pl.BlockSpec((1, n, dpad), lambda j: (j, 0, 0)),
    ]
    for (w, b, g, be) in layers:
        args += [w, b.reshape(-1, 1), g.reshape(1, -1), be.reshape(1, -1)]
        in_specs += [
            pl.BlockSpec(w.shape, lambda j: (0, 0)),
            pl.BlockSpec((w.shape[1], 1), lambda j: (0, 0)),
            pl.BlockSpec((1, w.shape[1]), lambda j: (0, 0)),
            pl.BlockSpec((1, w.shape[1]), lambda j: (0, 0)),
        ]
    scratch = [pltpu.VMEM((dpad, n), jnp.float32)]
    scratch += [pltpu.VMEM((k * d, n), jnp.float32) for d in dims]
    return pl.pallas_call(
        functools.partial(_fconv_body, k=k, dpad=dpad, dims=dims, nk=nk,
                          n=n, nlayer=nlayer),
        grid=(k,),
        in_specs=in_specs,
        out_specs=pl.BlockSpec((n, dims[-1]), lambda j: (0, 0)),
        out_shape=jax.ShapeDtypeStruct((n, dims[-1]), jnp.float32),
        scratch_shapes=scratch,
        interpret=_interpret,
    )(*args)


def _head_body(z_ref, s_ref, gl_ref, bel_ref, bc_ref,
               w1_ref, b1_ref, g1_ref, be1_ref,
               w2_ref, b2_ref, g2_ref, be2_ref, cw_ref, cb_ref, o_ref,
               *, nseg, kk, nk):
    s = jnp.sum(s_ref[...], axis=0)
    m = s[0:1] / nk
    v = s[1:2] / nk - m * m
    gl = gl_ref[...]
    bel = bel_ref[...]
    x = None
    for j in range(kk):
        zp = z_ref[:, j, :]
        xp = gl * (zp - m) / jnp.sqrt(v + EPS) + bel
        x = xp if x is None else jnp.maximum(x, xp)
    for (w_ref, b_ref, g_ref, be_ref) in ((w1_ref, b1_ref, g1_ref, be1_ref),
                                          (w2_ref, b2_ref, g2_ref, be2_ref)):
        z = jnp.maximum(
            jnp.dot(x.astype(jnp.bfloat16), w_ref[...].astype(jnp.bfloat16),
                    preferred_element_type=jnp.float32)
            + b_ref[...], 0.0)
        mz = jnp.mean(z, axis=0, keepdims=True)
        vz = jnp.mean((z - mz) * (z - mz), axis=0, keepdims=True)
        x = g_ref[...] * (z - mz) / jnp.sqrt(vz + EPS) + be_ref[...]
    bc = bc_ref[...]
    segs = []
    for sid in range(nseg):
        msk = bc == sid
        segs.append(jnp.max(jnp.where(msk, x, -jnp.inf), axis=0,
                            keepdims=True))
    gpool = jnp.concatenate(segs, axis=0)
    logits = (jnp.dot(gpool.astype(jnp.bfloat16),
                      cw_ref[...].astype(jnp.bfloat16),
                      preferred_element_type=jnp.float32)
              + cb_ref[...])
    mx = jnp.max(logits, axis=1, keepdims=True)
    shifted = logits - mx
    lse = jnp.log(jnp.sum(jnp.exp(shifted), axis=1, keepdims=True))
    o_ref[...] = shifted - lse


def _head(z, s, gl, bel, kk, bcol, mlp_params, cls_w, cls_b, nseg):
    (w1, b1, g1, be1), (w2, b2, g2, be2) = mlp_params
    n = z.shape[0]
    ncls = cls_w.shape[1]
    nk = float(n * kk)
    return pl.pallas_call(
        functools.partial(_head_body, nseg=nseg, kk=kk, nk=nk),
        out_shape=jax.ShapeDtypeStruct((nseg, ncls), jnp.float32),
        interpret=_interpret,
    )(z, s, gl.reshape(1, -1), bel.reshape(1, -1), bcol,
      w1, b1.reshape(1, -1), g1.reshape(1, -1), be1.reshape(1, -1),
      w2, b2.reshape(1, -1), g2.reshape(1, -1), be2.reshape(1, -1),
      cls_w, cls_b.reshape(1, -1))


def kernel(pos, batch, conv1_params, conv2_params, mlp_params, cls_W, cls_b):
    n = pos.shape[0]
    batch = batch.astype(jnp.int32)
    bcol = batch.reshape(n, 1)
    brow = batch.reshape(1, n)
    x1 = _dyn_edge_conv(pos, batch, bcol, brow, conv1_params, 20)
    z2, s2, g2, be2 = _dyn_edge_conv(x1, batch, bcol, brow, conv2_params, 10,
                                     last_raw=True)
    return _head(z2, s2, g2, be2, 10, bcol, mlp_params, cls_W, cls_b, 8)


# speedup vs baseline: 1.3378x; 1.1296x over previous
import functools

import jax
import jax.numpy as jnp
from jax import lax
from jax.experimental import pallas as pl
from jax.experimental.pallas import tpu as pltpu
from jax.experimental.pallas import tpu_sc as plsc

BIG = 1e30
BIG2 = 2e30
ROWS = 512
QB = 512
EPS = 1e-5

_interpret = False
_diag_ref_knn = False
_diag_jnp_gather = False


def _knn_body(b_smem, xa_ref, xb_ref, ba_ref, bb_ref, idx_ref,
              r_scr, base_scr, *, k, n, cw):
    nchunks = n // cw
    i = pl.program_id(0)
    q0 = i * QB
    qlo = b_smem[q0]
    qhi = b_smem[q0 + QB - 1]
    xb = xb_ref[...]
    bb = bb_ref[...]

    def compute_chunk(c, off):
        xa_c = xa_ref[pl.ds(c * cw, cw), :]
        ba_c = ba_ref[pl.ds(c * cw, cw), :]
        sqa = jnp.sum(xa_c * xa_c, axis=1, keepdims=True)
        dots = lax.dot_general(
            xa_c.astype(jnp.bfloat16), xb.astype(jnp.bfloat16),
            (((1,), (1,)), ((), ())),
            preferred_element_type=jnp.float32)
        r = sqa - 2.0 * dots
        r = jnp.where(ba_c != bb, BIG, r)
        r_scr[pl.ds(off * cw, cw), :] = r
        base_scr[off] = c * cw

    compute_chunk(0, 0)
    off = jnp.int32(1)
    for c in range(1, nchunks):
        clo = b_smem[c * cw]
        chi = b_smem[c * cw + cw - 1]
        act = jnp.logical_and(clo <= qhi, chi >= qlo)

        @pl.when(act)
        def _(c=c, off=off):
            compute_chunk(c, off)

        off = off + act.astype(jnp.int32)
    n_act = off

    iota_c = lax.broadcasted_iota(jnp.int32, (cw, QB), 0)
    rows = []
    for _ in range(k):
        def pass1(ci, cur):
            blk = r_scr[pl.ds(ci * cw, cw), :]
            return jnp.minimum(cur, jnp.min(blk, axis=0, keepdims=True))
        m = lax.fori_loop(0, n_act, pass1,
                          jnp.full((1, QB), BIG, jnp.float32))

        def pass2(ci, cur):
            blk = r_scr[pl.ds(ci * cw, cw), :]
            gx = iota_c + base_scr[ci]
            cand = jnp.min(jnp.where(blk == m, gx, n), axis=0, keepdims=True)
            return jnp.minimum(cur, cand)
        j = lax.fori_loop(0, n_act, pass2,
                          jnp.full((1, QB), n, jnp.int32))
        rows.append(j)

        def pass3(ci, _):
            blk = r_scr[pl.ds(ci * cw, cw), :]
            gx = iota_c + base_scr[ci]
            r_scr[pl.ds(ci * cw, cw), :] = jnp.where(
                (blk == m) & (gx == j), BIG2, blk)
            return 0
        lax.fori_loop(0, n_act, pass3, 0)
    idx_ref[...] = jnp.concatenate(rows, axis=0)


def _knn(x, batch1d, bcol, brow, k):
    n, d = x.shape
    cw = 512
    grid_spec = pltpu.PrefetchScalarGridSpec(
        num_scalar_prefetch=1,
        grid=(n // QB,),
        in_specs=[
            pl.BlockSpec((n, d), lambda i, b: (0, 0)),
            pl.BlockSpec((QB, d), lambda i, b: (i, 0)),
            pl.BlockSpec((n, 1), lambda i, b: (0, 0)),
            pl.BlockSpec((1, QB), lambda i, b: (0, i)),
        ],
        out_specs=pl.BlockSpec((k, QB), lambda i, b: (0, i)),
        scratch_shapes=[
            pltpu.VMEM((n, QB), jnp.float32),
            pltpu.SMEM((n // cw,), jnp.int32),
        ],
    )
    return pl.pallas_call(
        functools.partial(_knn_body, k=k, n=n, cw=cw),
        grid_spec=grid_spec,
        out_shape=jax.ShapeDtypeStruct((k, n), jnp.int32),
        compiler_params=pltpu.CompilerParams(
            dimension_semantics=("parallel",)),
        interpret=_interpret,
    )(batch1d, x, x, bcol, brow)


def _sc_gather(table, idx):
    _, dcols = table.shape
    (b_total,) = idx.shape
    info = plsc.get_sparse_core_info()
    nw = info.num_cores * info.num_subcores
    b_per_w = b_total // nw
    mesh = plsc.VectorSubcoreMesh(core_axis_name="c", subcore_axis_name="s")

    @functools.partial(
        pl.kernel, mesh=mesh,
        out_type=jax.ShapeDtypeStruct((b_total, dcols), jnp.float32),
        compiler_params=pltpu.CompilerParams(use_tc_tiling_on_sc=False),
        scratch_types=[
            pltpu.VMEM((b_per_w,), jnp.int32),
            pltpu.VMEM((b_per_w, dcols), jnp.float32),
            pltpu.SemaphoreType.DMA,
        ],
    )
    def gk(table_hbm, idx_hbm, out_hbm, idx_v, rows_v, sem):
        wid = lax.axis_index("s") * info.num_cores + lax.axis_index("c")
        base = wid * b_per_w
        pltpu.sync_copy(idx_hbm.at[pl.ds(base, b_per_w)], idx_v)
        pltpu.async_copy(table_hbm.at[idx_v], rows_v, sem).wait()
        pltpu.sync_copy(rows_v, out_hbm.at[pl.ds(base, b_per_w)])

    return gk(table, idx)


def _accum_stats(s_ref, y):
    ps = jnp.sum(y, axis=0, keepdims=True)
    pss = jnp.sum(y * y, axis=0, keepdims=True)
    s_ref[...] = jnp.concatenate([ps, pss], axis=0)[None]


def _edge_l1_body(x_ref, g_ref, w_ref, b_ref, z_ref, s_ref, *, k):
    xi = x_ref[...]
    xj = g_ref[...]
    r_rows, _, d = xj.shape
    feat = jnp.concatenate(
        [jnp.broadcast_to(xi[:, None, :], xj.shape), xj - xi[:, None, :]],
        axis=2).reshape(r_rows * k, 2 * d)
    z = jnp.maximum(
        jnp.dot(feat.astype(jnp.bfloat16), w_ref[...].astype(jnp.bfloat16),
                preferred_element_type=jnp.float32) + b_ref[...], 0.0)
    d1 = z.shape[1]
    z_ref[...] = z.reshape(r_rows, k, d1)
    _accum_stats(s_ref, z)


def _edge_l1(x, g3, w1, b1, k):
    n, _, d = g3.shape
    d1 = w1.shape[1]
    return pl.pallas_call(
        functools.partial(_edge_l1_body, k=k),
        grid=(n // ROWS,),
        in_specs=[
            pl.BlockSpec((ROWS, d), lambda i: (i, 0)),
            pl.BlockSpec((ROWS, k, d), lambda i: (i, 0, 0)),
            pl.BlockSpec(w1.shape, lambda i: (0, 0)),
            pl.BlockSpec((1, d1), lambda i: (0, 0)),
        ],
        out_specs=[
            pl.BlockSpec((ROWS, k, d1), lambda i: (i, 0, 0)),
            pl.BlockSpec((1, 2, d1), lambda i: (i, 0, 0)),
        ],
        out_shape=[jax.ShapeDtypeStruct((n, k, d1), jnp.float32),
                   jax.ShapeDtypeStruct((n // ROWS, 2, d1), jnp.float32)],
        compiler_params=pltpu.CompilerParams(
            dimension_semantics=("parallel",)),
        interpret=_interpret,
    )(x, g3, w1, b1)


def _norm(z2d, s_ref, g_ref, be_ref, nk):
    s = jnp.sum(s_ref[...], axis=0)
    m = s[0:1] / nk
    v = s[1:2] / nk - m * m
    return g_ref[...] * (z2d - m) / jnp.sqrt(v + EPS) + be_ref[...]


def _edge_next_body(z_ref, s_ref, g_ref, be_ref, w_ref, b_ref,
                    zo_ref, so_ref, *, k, nk):
    z = z_ref[...]
    r_rows, _, d1 = z.shape
    x = _norm(z.reshape(r_rows * k, d1), s_ref, g_ref, be_ref, nk)
    y = jnp.maximum(
        jnp.dot(x.astype(jnp.bfloat16), w_ref[...].astype(jnp.bfloat16),
                preferred_element_type=jnp.float32)
        + b_ref[...], 0.0)
    d2 = y.shape[1]
    zo_ref[...] = y.reshape(r_rows, k, d2)
    _accum_stats(so_ref, y)


def _edge_next(z, s, g, be, w, b, k):
    n, _, d1 = z.shape
    d2 = w.shape[1]
    nk = float(n * k)
    return pl.pallas_call(
        functools.partial(_edge_next_body, k=k, nk=nk),
        grid=(n // ROWS,),
        in_specs=[
            pl.BlockSpec((ROWS, k, d1), lambda i: (i, 0, 0)),
            pl.BlockSpec(s.shape, lambda i: (0, 0, 0)),
            pl.BlockSpec((1, d1), lambda i: (0, 0)),
            pl.BlockSpec((1, d1), lambda i: (0, 0)),
            pl.BlockSpec((d1, d2), lambda i: (0, 0)),
            pl.BlockSpec((1, d2), lambda i: (0, 0)),
        ],
        out_specs=[
            pl.BlockSpec((ROWS, k, d2), lambda i: (i, 0, 0)),
            pl.BlockSpec((1, 2, d2), lambda i: (i, 0, 0)),
        ],
        out_shape=[jax.ShapeDtypeStruct((n, k, d2), jnp.float32),
                   jax.ShapeDtypeStruct((n // ROWS, 2, d2), jnp.float32)],
        compiler_params=pltpu.CompilerParams(
            dimension_semantics=("parallel",)),
        interpret=_interpret,
    )(z, s, g, be, w, b)


def _edge_final_body(z_ref, s_ref, g_ref, be_ref, o_ref, *, k, nk):
    z = z_ref[...]
    r_rows, _, dl = z.shape
    x = _norm(z.reshape(r_rows * k, dl), s_ref, g_ref, be_ref,
              nk).reshape(r_rows, k, dl)
    zmax = x[:, 0, :]
    for j in range(1, k):
        zmax = jnp.maximum(zmax, x[:, j, :])
    o_ref[...] = zmax


def _edge_final(z, s, g, be, k):
    n, _, dl = z.shape
    nk = float(n * k)
    return pl.pallas_call(
        functools.partial(_edge_final_body, k=k, nk=nk),
        grid=(n // ROWS,),
        in_specs=[
            pl.BlockSpec((ROWS, k, dl), lambda i: (i, 0, 0)),
            pl.BlockSpec(s.shape, lambda i: (0, 0, 0)),
            pl.BlockSpec((1, dl), lambda i: (0, 0)),
            pl.BlockSpec((1, dl), lambda i: (0, 0)),
        ],
        out_specs=pl.BlockSpec((ROWS, dl), lambda i: (i, 0)),
        out_shape=jax.ShapeDtypeStruct((n, dl), jnp.float32),
        compiler_params=pltpu.CompilerParams(
            dimension_semantics=("parallel",)),
        interpret=_interpret,
    )(z, s, g, be)


def _dyn_edge_conv(x, batch1d, bcol, brow, params, k, last_raw=False):
    n, _ = x.shape
    if _diag_ref_knn:
        sq = jnp.sum(x * x, axis=1)
        dist = sq[:, None] + sq[None, :] - 2.0 * (x @ x.T)
        dist = jnp.where(bcol != brow, jnp.inf, dist)
        _, idx_d = lax.top_k(-dist, k)
        idx_t = idx_d.T
    else:
        idx_t = _knn(x, batch1d, bcol, brow, k)
    w1, b1, g1, be1 = params[0]
    d = x.shape[1]
    dpad = max(16, -(-d // 16) * 16)
    if dpad != d:
        xg = jnp.pad(x, ((0, 0), (0, dpad - d)))
        w1 = jnp.concatenate([
            jnp.pad(w1[:d], ((0, dpad - d), (0, 0))),
            jnp.pad(w1[d:], ((0, dpad - d), (0, 0)))], axis=0)
    else:
        xg = x
    if not last_raw:
        if _diag_jnp_gather:
            gathered = xg[idx_t.reshape(k * n)]
        else:
            gathered = _sc_gather(xg, idx_t.reshape(k * n))
        layers = [(w1, b1, g1, be1)] + list(params[1:])
        return _fused_conv(xg, gathered.reshape(k, n, dpad), layers, k)
    if _diag_jnp_gather:
        gathered = xg[idx_t.T.reshape(n * k)]
    else:
        gathered = _sc_gather(xg, idx_t.T.reshape(n * k))
    z, s = _edge_l1(xg, gathered.reshape(n, k, dpad), w1,
                    b1.reshape(1, -1), k)
    gp, bep = g1.reshape(1, -1), be1.reshape(1, -1)
    for (w, b, g, be) in params[1:]:
        z, s = _edge_next(z, s, gp, bep, w, b.reshape(1, -1), k)
        gp, bep = g.reshape(1, -1), be.reshape(1, -1)
    if last_raw:
        return z, s, gp, bep
    return _edge_final(z, s, gp, bep, k)


def _fconv_body(x_ref, g_ref, *refs, k, dpad, dims, nk, n, nlayer):
    lrefs = [refs[4 * l:4 * l + 4] for l in range(nlayer)]
    o_ref = refs[4 * nlayer]
    xt_scr = refs[4 * nlayer + 1]
    z_scrs = refs[4 * nlayer + 2:4 * nlayer + 2 + nlayer]
    j = pl.program_id(0)
    d1 = dims[0]

    @pl.when(j == 0)
    def _():
        xt_scr[...] = jnp.transpose(x_ref[...])

    xt = xt_scr[...]
    xjt = jnp.transpose(g_ref[0])
    featt = jnp.concatenate([xt, xjt - xt], axis=0)
    w1t = jnp.transpose(lrefs[0][0][...])
    z1 = jnp.maximum(
        lax.dot_general(w1t.astype(jnp.bfloat16), featt.astype(jnp.bfloat16),
                        (((1,), (0,)), ((), ())),
                        preferred_element_type=jnp.float32)
        + lrefs[0][1][...], 0.0)
    z_scrs[0][pl.ds(j * d1, d1), :] = z1

    @pl.when(j == k - 1)
    def _():
        zin = z_scrs[0]
        din = d1
        for l in range(1, nlayer + 1):
            s = jnp.zeros((din, 1), jnp.float32)
            ss = jnp.zeros((din, 1), jnp.float32)
            for jj in range(k):
                zp = zin[jj * din:(jj + 1) * din, :]
                s = s + jnp.sum(zp, axis=1, keepdims=True)
                ss = ss + jnp.sum(zp * zp, axis=1, keepdims=True)
            m = s / nk
            v = ss / nk - m * m
            gcol = jnp.transpose(lrefs[l - 1][2][...])
            becol = jnp.transpose(lrefs[l - 1][3][...])
            inv = jnp.sqrt(v + EPS)
            if l < nlayer:
                dout = dims[l]
                wt = jnp.transpose(lrefs[l][0][...]).astype(jnp.bfloat16)
                bcol = lrefs[l][1][...]
                for jj in range(k):
                    xn = gcol * (zin[jj * din:(jj + 1) * din, :] - m) / inv \
                        + becol
                    zl = jnp.maximum(
                        lax.dot_general(wt, xn.astype(jnp.bfloat16),
                                        (((1,), (0,)), ((), ())),
                                        preferred_element_type=jnp.float32)
                        + bcol, 0.0)
                    z_scrs[l][pl.ds(jj * dout, dout), :] = zl
                zin = z_scrs[l]
                din = dout
            else:
                acc = None
                for jj in range(k):
                    xn = gcol * (zin[jj * din:(jj + 1) * din, :] - m) / inv \
                        + becol
                    acc = xn if acc is None else jnp.maximum(acc, xn)
                o_ref[...] = jnp.transpose(acc)


def _fused_conv(x, gathered_knd, layers, k):
    n, dpad = x.shape
    nlayer = len(layers)
    dims = [w.shape[1] for (w, _, _, _) in layers]
    nk = float(n * k)
    args = [x, gathered_knd]
    in_specs = [
        pl.BlockSpec((n, dpad), lambda j: (0, 0)),
        pl.BlockSpec((1, n, dpad), lambda j: (j, 0, 0)),
    ]
    for (w, b, g, be) in layers:
        args += [w, b.reshape(-1, 1), g.reshape(1, -1), be.reshape(1, -1)]
        in_specs += [
            pl.BlockSpec(w.shape, lambda j: (0, 0)),
            pl.BlockSpec((w.shape[1], 1), lambda j: (0, 0)),
            pl.BlockSpec((1, w.shape[1]), lambda j: (0, 0)),
            pl.BlockSpec((1, w.shape[1]), lambda j: (0, 0)),
        ]
    scratch = [pltpu.VMEM((dpad, n), jnp.float32)]
    scratch += [pltpu.VMEM((k * d, n), jnp.float32) for d in dims]
    return pl.pallas_call(
        functools.partial(_fconv_body, k=k, dpad=dpad, dims=dims, nk=nk,
                          n=n, nlayer=nlayer),
        grid=(k,),
        in_specs=in_specs,
        out_specs=pl.BlockSpec((n, dims[-1]), lambda j: (0, 0)),
        out_shape=jax.ShapeDtypeStruct((n, dims[-1]), jnp.float32),
        scratch_shapes=scratch,
        interpret=_interpret,
    )(*args)


def _head_body(x_ref, bc_ref, w1_ref, b1_ref, g1_ref, be1_ref,
               w2_ref, b2_ref, g2_ref, be2_ref, cw_ref, cb_ref, o_ref,
               *, nseg):
    x = x_ref[...]
    for (w_ref, b_ref, g_ref, be_ref) in ((w1_ref, b1_ref, g1_ref, be1_ref),
                                          (w2_ref, b2_ref, g2_ref, be2_ref)):
        z = jnp.maximum(
            jnp.dot(x.astype(jnp.bfloat16), w_ref[...].astype(jnp.bfloat16),
                    preferred_element_type=jnp.float32)
            + b_ref[...], 0.0)
        mz = jnp.mean(z, axis=0, keepdims=True)
        vz = jnp.mean((z - mz) * (z - mz), axis=0, keepdims=True)
        x = g_ref[...] * (z - mz) / jnp.sqrt(vz + EPS) + be_ref[...]
    bc = bc_ref[...]
    segs = []
    for sid in range(nseg):
        msk = bc == sid
        segs.append(jnp.max(jnp.where(msk, x, -jnp.inf), axis=0,
                            keepdims=True))
    gpool = jnp.concatenate(segs, axis=0)
    logits = (jnp.dot(gpool.astype(jnp.bfloat16),
                      cw_ref[...].astype(jnp.bfloat16),
                      preferred_element_type=jnp.float32)
              + cb_ref[...])
    mx = jnp.max(logits, axis=1, keepdims=True)
    shifted = logits - mx
    lse = jnp.log(jnp.sum(jnp.exp(shifted), axis=1, keepdims=True))
    o_ref[...] = shifted - lse


def _head(x, bcol, mlp_params, cls_w, cls_b, nseg):
    (w1, b1, g1, be1), (w2, b2, g2, be2) = mlp_params
    ncls = cls_w.shape[1]
    return pl.pallas_call(
        functools.partial(_head_body, nseg=nseg),
        out_shape=jax.ShapeDtypeStruct((nseg, ncls), jnp.float32),
        interpret=_interpret,
    )(x, bcol, w1, b1.reshape(1, -1), g1.reshape(1, -1), be1.reshape(1, -1),
      w2, b2.reshape(1, -1), g2.reshape(1, -1), be2.reshape(1, -1),
      cls_w, cls_b.reshape(1, -1))


def kernel(pos, batch, conv1_params, conv2_params, mlp_params, cls_W, cls_b):
    n = pos.shape[0]
    batch = batch.astype(jnp.int32)
    bcol = batch.reshape(n, 1)
    brow = batch.reshape(1, n)
    x1 = _dyn_edge_conv(pos, batch, bcol, brow, conv1_params, 20)
    x2 = _dyn_edge_conv(x1, batch, bcol, brow, conv2_params, 10)
    return _head(x2, bcol, mlp_params, cls_W, cls_b, 8)


# speedup vs baseline: 1.4703x; 1.0990x over previous
import functools

import jax
import jax.numpy as jnp
from jax import lax
from jax.experimental import pallas as pl
from jax.experimental.pallas import tpu as pltpu
from jax.experimental.pallas import tpu_sc as plsc

BIG = 1e30
BIG2 = 2e30
ROWS = 512
QB = 512
EPS = 1e-5

_interpret = False
_diag_ref_knn = False
_diag_jnp_gather = False


def _knn_body(b_smem, xa_ref, xb_ref, ba_ref, bb_ref, idx_ref,
              r_scr, base_scr, *, k, n, cw):
    nchunks = n // cw
    i = pl.program_id(0)
    q0 = i * QB
    qlo = b_smem[q0]
    qhi = b_smem[q0 + QB - 1]
    xb = xb_ref[...]
    bb = bb_ref[...]

    def compute_chunk(c, off):
        xa_c = xa_ref[pl.ds(c * cw, cw), :]
        ba_c = ba_ref[pl.ds(c * cw, cw), :]
        sqa = jnp.sum(xa_c * xa_c, axis=1, keepdims=True)
        dots = lax.dot_general(
            xa_c.astype(jnp.bfloat16), xb.astype(jnp.bfloat16),
            (((1,), (1,)), ((), ())),
            preferred_element_type=jnp.float32)
        r = sqa - 2.0 * dots
        r = jnp.where(ba_c != bb, BIG, r)
        r_scr[pl.ds(off * cw, cw), :] = r
        base_scr[off] = c * cw

    compute_chunk(0, 0)
    off = jnp.int32(1)
    for c in range(1, nchunks):
        clo = b_smem[c * cw]
        chi = b_smem[c * cw + cw - 1]
        act = jnp.logical_and(clo <= qhi, chi >= qlo)

        @pl.when(act)
        def _(c=c, off=off):
            compute_chunk(c, off)

        off = off + act.astype(jnp.int32)
    n_act = off

    iota_c = lax.broadcasted_iota(jnp.int32, (cw, QB), 0)
    rows = []
    m_prev = None
    j_prev = None
    for _ in range(k):
        def pass1(ci, cur, m_prev=m_prev, j_prev=j_prev):
            blk = r_scr[pl.ds(ci * cw, cw), :]
            if m_prev is not None:
                gx = iota_c + base_scr[ci]
                blk = jnp.where((blk == m_prev) & (gx == j_prev), BIG2, blk)
                r_scr[pl.ds(ci * cw, cw), :] = blk
            return jnp.minimum(cur, jnp.min(blk, axis=0, keepdims=True))
        m = lax.fori_loop(0, n_act, pass1,
                          jnp.full((1, QB), BIG, jnp.float32))

        def pass2(ci, cur, m=m):
            blk = r_scr[pl.ds(ci * cw, cw), :]
            gx = iota_c + base_scr[ci]
            cand = jnp.min(jnp.where(blk == m, gx, n), axis=0, keepdims=True)
            return jnp.minimum(cur, cand)
        j = lax.fori_loop(0, n_act, pass2,
                          jnp.full((1, QB), n, jnp.int32))
        rows.append(j)
        m_prev = m
        j_prev = j
    idx_ref[...] = jnp.concatenate(rows, axis=0)


def _knn(x, batch1d, bcol, brow, k):
    n, d = x.shape
    cw = 512
    grid_spec = pltpu.PrefetchScalarGridSpec(
        num_scalar_prefetch=1,
        grid=(n // QB,),
        in_specs=[
            pl.BlockSpec((n, d), lambda i, b: (0, 0)),
            pl.BlockSpec((QB, d), lambda i, b: (i, 0)),
            pl.BlockSpec((n, 1), lambda i, b: (0, 0)),
            pl.BlockSpec((1, QB), lambda i, b: (0, i)),
        ],
        out_specs=pl.BlockSpec((k, QB), lambda i, b: (0, i)),
        scratch_shapes=[
            pltpu.VMEM((n, QB), jnp.float32),
            pltpu.SMEM((n // cw,), jnp.int32),
        ],
    )
    return pl.pallas_call(
        functools.partial(_knn_body, k=k, n=n, cw=cw),
        grid_spec=grid_spec,
        out_shape=jax.ShapeDtypeStruct((k, n), jnp.int32),
        compiler_params=pltpu.CompilerParams(
            dimension_semantics=("parallel",)),
        interpret=_interpret,
    )(batch1d, x, x, bcol, brow)


def _sc_gather(table, idx):
    _, dcols = table.shape
    (b_total,) = idx.shape
    info = plsc.get_sparse_core_info()
    nw = info.num_cores * info.num_subcores
    b_per_w = b_total // nw
    mesh = plsc.VectorSubcoreMesh(core_axis_name="c", subcore_axis_name="s")

    @functools.partial(
        pl.kernel, mesh=mesh,
        out_type=jax.ShapeDtypeStruct((b_total, dcols), jnp.float32),
        compiler_params=pltpu.CompilerParams(use_tc_tiling_on_sc=False),
        scratch_types=[
            pltpu.VMEM((b_per_w,), jnp.int32),
            pltpu.VMEM((b_per_w, dcols), jnp.float32),
            pltpu.SemaphoreType.DMA,
        ],
    )
    def gk(table_hbm, idx_hbm, out_hbm, idx_v, rows_v, sem):
        wid = lax.axis_index("s") * info.num_cores + lax.axis_index("c")
        base = wid * b_per_w
        pltpu.sync_copy(idx_hbm.at[pl.ds(base, b_per_w)], idx_v)
        pltpu.async_copy(table_hbm.at[idx_v], rows_v, sem).wait()
        pltpu.sync_copy(rows_v, out_hbm.at[pl.ds(base, b_per_w)])

    return gk(table, idx)


def _accum_stats(s_ref, y):
    ps = jnp.sum(y, axis=0, keepdims=True)
    pss = jnp.sum(y * y, axis=0, keepdims=True)
    s_ref[...] = jnp.concatenate([ps, pss], axis=0)[None]


def _edge_l1_body(x_ref, g_ref, w_ref, b_ref, z_ref, s_ref, *, k):
    xi = x_ref[...]
    xj = g_ref[...]
    r_rows, _, d = xj.shape
    feat = jnp.concatenate(
        [jnp.broadcast_to(xi[:, None, :], xj.shape), xj - xi[:, None, :]],
        axis=2).reshape(r_rows * k, 2 * d)
    z = jnp.maximum(
        jnp.dot(feat.astype(jnp.bfloat16), w_ref[...].astype(jnp.bfloat16),
                preferred_element_type=jnp.float32) + b_ref[...], 0.0)
    d1 = z.shape[1]
    z_ref[...] = z.reshape(r_rows, k, d1)
    _accum_stats(s_ref, z)


def _edge_l1(x, g3, w1, b1, k):
    n, _, d = g3.shape
    d1 = w1.shape[1]
    return pl.pallas_call(
        functools.partial(_edge_l1_body, k=k),
        grid=(n // ROWS,),
        in_specs=[
            pl.BlockSpec((ROWS, d), lambda i: (i, 0)),
            pl.BlockSpec((ROWS, k, d), lambda i: (i, 0, 0)),
            pl.BlockSpec(w1.shape, lambda i: (0, 0)),
            pl.BlockSpec((1, d1), lambda i: (0, 0)),
        ],
        out_specs=[
            pl.BlockSpec((ROWS, k, d1), lambda i: (i, 0, 0)),
            pl.BlockSpec((1, 2, d1), lambda i: (i, 0, 0)),
        ],
        out_shape=[jax.ShapeDtypeStruct((n, k, d1), jnp.float32),
                   jax.ShapeDtypeStruct((n // ROWS, 2, d1), jnp.float32)],
        compiler_params=pltpu.CompilerParams(
            dimension_semantics=("parallel",)),
        interpret=_interpret,
    )(x, g3, w1, b1)


def _norm(z2d, s_ref, g_ref, be_ref, nk):
    s = jnp.sum(s_ref[...], axis=0)
    m = s[0:1] / nk
    v = s[1:2] / nk - m * m
    return g_ref[...] * (z2d - m) / jnp.sqrt(v + EPS) + be_ref[...]


def _edge_next_body(z_ref, s_ref, g_ref, be_ref, w_ref, b_ref,
                    zo_ref, so_ref, *, k, nk):
    z = z_ref[...]
    r_rows, _, d1 = z.shape
    x = _norm(z.reshape(r_rows * k, d1), s_ref, g_ref, be_ref, nk)
    y = jnp.maximum(
        jnp.dot(x.astype(jnp.bfloat16), w_ref[...].astype(jnp.bfloat16),
                preferred_element_type=jnp.float32)
        + b_ref[...], 0.0)
    d2 = y.shape[1]
    zo_ref[...] = y.reshape(r_rows, k, d2)
    _accum_stats(so_ref, y)


def _edge_next(z, s, g, be, w, b, k):
    n, _, d1 = z.shape
    d2 = w.shape[1]
    nk = float(n * k)
    return pl.pallas_call(
        functools.partial(_edge_next_body, k=k, nk=nk),
        grid=(n // ROWS,),
        in_specs=[
            pl.BlockSpec((ROWS, k, d1), lambda i: (i, 0, 0)),
            pl.BlockSpec(s.shape, lambda i: (0, 0, 0)),
            pl.BlockSpec((1, d1), lambda i: (0, 0)),
            pl.BlockSpec((1, d1), lambda i: (0, 0)),
            pl.BlockSpec((d1, d2), lambda i: (0, 0)),
            pl.BlockSpec((1, d2), lambda i: (0, 0)),
        ],
        out_specs=[
            pl.BlockSpec((ROWS, k, d2), lambda i: (i, 0, 0)),
            pl.BlockSpec((1, 2, d2), lambda i: (i, 0, 0)),
        ],
        out_shape=[jax.ShapeDtypeStruct((n, k, d2), jnp.float32),
                   jax.ShapeDtypeStruct((n // ROWS, 2, d2), jnp.float32)],
        compiler_params=pltpu.CompilerParams(
            dimension_semantics=("parallel",)),
        interpret=_interpret,
    )(z, s, g, be, w, b)


def _edge_final_body(z_ref, s_ref, g_ref, be_ref, o_ref, *, k, nk):
    z = z_ref[...]
    r_rows, _, dl = z.shape
    x = _norm(z.reshape(r_rows * k, dl), s_ref, g_ref, be_ref,
              nk).reshape(r_rows, k, dl)
    zmax = x[:, 0, :]
    for j in range(1, k):
        zmax = jnp.maximum(zmax, x[:, j, :])
    o_ref[...] = zmax


def _edge_final(z, s, g, be, k):
    n, _, dl = z.shape
    nk = float(n * k)
    return pl.pallas_call(
        functools.partial(_edge_final_body, k=k, nk=nk),
        grid=(n // ROWS,),
        in_specs=[
            pl.BlockSpec((ROWS, k, dl), lambda i: (i, 0, 0)),
            pl.BlockSpec(s.shape, lambda i: (0, 0, 0)),
            pl.BlockSpec((1, dl), lambda i: (0, 0)),
            pl.BlockSpec((1, dl), lambda i: (0, 0)),
        ],
        out_specs=pl.BlockSpec((ROWS, dl), lambda i: (i, 0)),
        out_shape=jax.ShapeDtypeStruct((n, dl), jnp.float32),
        compiler_params=pltpu.CompilerParams(
            dimension_semantics=("parallel",)),
        interpret=_interpret,
    )(z, s, g, be)


def _dyn_edge_conv(x, batch1d, bcol, brow, params, k, last_raw=False):
    n, _ = x.shape
    if _diag_ref_knn:
        sq = jnp.sum(x * x, axis=1)
        dist = sq[:, None] + sq[None, :] - 2.0 * (x @ x.T)
        dist = jnp.where(bcol != brow, jnp.inf, dist)
        _, idx_d = lax.top_k(-dist, k)
        idx_t = idx_d.T
    else:
        idx_t = _knn(x, batch1d, bcol, brow, k)
    w1, b1, g1, be1 = params[0]
    d = x.shape[1]
    dpad = max(16, -(-d // 16) * 16)
    if dpad != d:
        xg = jnp.pad(x, ((0, 0), (0, dpad - d)))
        w1 = jnp.concatenate([
            jnp.pad(w1[:d], ((0, dpad - d), (0, 0))),
            jnp.pad(w1[d:], ((0, dpad - d), (0, 0)))], axis=0)
    else:
        xg = x
    if not last_raw:
        if _diag_jnp_gather:
            gathered = xg[idx_t.reshape(k * n)]
        else:
            gathered = _sc_gather(xg, idx_t.reshape(k * n))
        layers = [(w1, b1, g1, be1)] + list(params[1:])
        return _fused_conv(xg, gathered.reshape(k, n, dpad), layers, k)
    if _diag_jnp_gather:
        gathered = xg[idx_t.T.reshape(n * k)]
    else:
        gathered = _sc_gather(xg, idx_t.T.reshape(n * k))
    z, s = _edge_l1(xg, gathered.reshape(n, k, dpad), w1,
                    b1.reshape(1, -1), k)
    gp, bep = g1.reshape(1, -1), be1.reshape(1, -1)
    for (w, b, g, be) in params[1:]:
        z, s = _edge_next(z, s, gp, bep, w, b.reshape(1, -1), k)
        gp, bep = g.reshape(1, -1), be.reshape(1, -1)
    if last_raw:
        return z, s, gp, bep
    return _edge_final(z, s, gp, bep, k)


def _fconv_body(x_ref, g_ref, *refs, k, dpad, dims, nk, n, nlayer):
    lrefs = [refs[4 * l:4 * l + 4] for l in range(nlayer)]
    o_ref = refs[4 * nlayer]
    xt_scr = refs[4 * nlayer + 1]
    z_scrs = refs[4 * nlayer + 2:4 * nlayer + 2 + nlayer]
    j = pl.program_id(0)
    d1 = dims[0]

    @pl.when(j == 0)
    def _():
        xt_scr[...] = jnp.transpose(x_ref[...])

    xt = xt_scr[...]
    xjt = jnp.transpose(g_ref[0])
    featt = jnp.concatenate([xt, xjt - xt], axis=0)
    w1t = jnp.transpose(lrefs[0][0][...])
    z1 = jnp.maximum(
        lax.dot_general(w1t.astype(jnp.bfloat16), featt.astype(jnp.bfloat16),
                        (((1,), (0,)), ((), ())),
                        preferred_element_type=jnp.float32)
        + lrefs[0][1][...], 0.0)
    z_scrs[0][pl.ds(j * d1, d1), :] = z1

    @pl.when(j == k - 1)
    def _():
        zin = z_scrs[0]
        din = d1
        for l in range(1, nlayer + 1):
            s = jnp.zeros((din, 1), jnp.float32)
            ss = jnp.zeros((din, 1), jnp.float32)
            for jj in range(k):
                zp = zin[jj * din:(jj + 1) * din, :]
                s = s + jnp.sum(zp, axis=1, keepdims=True)
                ss = ss + jnp.sum(zp * zp, axis=1, keepdims=True)
            m = s / nk
            v = ss / nk - m * m
            gcol = jnp.transpose(lrefs[l - 1][2][...])
            becol = jnp.transpose(lrefs[l - 1][3][...])
            inv = jnp.sqrt(v + EPS)
            if l < nlayer:
                dout = dims[l]
                wt = jnp.transpose(lrefs[l][0][...]).astype(jnp.bfloat16)
                bcol = lrefs[l][1][...]
                for jj in range(k):
                    xn = gcol * (zin[jj * din:(jj + 1) * din, :] - m) / inv \
                        + becol
                    zl = jnp.maximum(
                        lax.dot_general(wt, xn.astype(jnp.bfloat16),
                                        (((1,), (0,)), ((), ())),
                                        preferred_element_type=jnp.float32)
                        + bcol, 0.0)
                    z_scrs[l][pl.ds(jj * dout, dout), :] = zl
                zin = z_scrs[l]
                din = dout
            else:
                acc = None
                for jj in range(k):
                    xn = gcol * (zin[jj * din:(jj + 1) * din, :] - m) / inv \
                        + becol
                    acc = xn if acc is None else jnp.maximum(acc, xn)
                o_ref[...] = jnp.transpose(acc)


def _fused_conv(x, gathered_knd, layers, k):
    n, dpad = x.shape
    nlayer = len(layers)
    dims = [w.shape[1] for (w, _, _, _) in layers]
    nk = float(n * k)
    args = [x, gathered_knd]
    in_specs = [
        pl.BlockSpec((n, dpad), lambda j: (0, 0)),
        pl.BlockSpec((1, n, dpad), lambda j: (j, 0, 0)),
    ]
    for (w, b, g, be) in layers:
        args += [w, b.reshape(-1, 1), g.reshape(1, -1), be.reshape(1, -1)]
        in_specs += [
            pl.BlockSpec(w.shape, lambda j: (0, 0)),
            pl.BlockSpec((w.shape[1], 1), lambda j: (0, 0)),
            pl.BlockSpec((1, w.shape[1]), lambda j: (0, 0)),
            pl.BlockSpec((1, w.shape[1]), lambda j: (0, 0)),
        ]
    scratch = [pltpu.VMEM((dpad, n), jnp.float32)]
    scratch += [pltpu.VMEM((k * d, n), jnp.float32) for d in dims]
    return pl.pallas_call(
        functools.partial(_fconv_body, k=k, dpad=dpad, dims=dims, nk=nk,
                          n=n, nlayer=nlayer),
        grid=(k,),
        in_specs=in_specs,
        out_specs=pl.BlockSpec((n, dims[-1]), lambda j: (0, 0)),
        out_shape=jax.ShapeDtypeStruct((n, dims[-1]), jnp.float32),
        scratch_shapes=scratch,
        interpret=_interpret,
    )(*args)


def _head_body(x_ref, bc_ref, w1_ref, b1_ref, g1_ref, be1_ref,
               w2_ref, b2_ref, g2_ref, be2_ref, cw_ref, cb_ref, o_ref,
               *, nseg):
    x = x_ref[...]
    for (w_ref, b_ref, g_ref, be_ref) in ((w1_ref, b1_ref, g1_ref, be1_ref),
                                          (w2_ref, b2_ref, g2_ref, be2_ref)):
        z = jnp.maximum(
            jnp.dot(x.astype(jnp.bfloat16), w_ref[...].astype(jnp.bfloat16),
                    preferred_element_type=jnp.float32)
            + b_ref[...], 0.0)
        mz = jnp.mean(z, axis=0, keepdims=True)
        vz = jnp.mean((z - mz) * (z - mz), axis=0, keepdims=True)
        x = g_ref[...] * (z - mz) / jnp.sqrt(vz + EPS) + be_ref[...]
    bc = bc_ref[...]
    segs = []
    for sid in range(nseg):
        msk = bc == sid
        segs.append(jnp.max(jnp.where(msk, x, -jnp.inf), axis=0,
                            keepdims=True))
    gpool = jnp.concatenate(segs, axis=0)
    logits = (jnp.dot(gpool.astype(jnp.bfloat16),
                      cw_ref[...].astype(jnp.bfloat16),
                      preferred_element_type=jnp.float32)
              + cb_ref[...])
    mx = jnp.max(logits, axis=1, keepdims=True)
    shifted = logits - mx
    lse = jnp.log(jnp.sum(jnp.exp(shifted), axis=1, keepdims=True))
    o_ref[...] = shifted - lse


def _head(x, bcol, mlp_params, cls_w, cls_b, nseg):
    (w1, b1, g1, be1), (w2, b2, g2, be2) = mlp_params
    ncls = cls_w.shape[1]
    return pl.pallas_call(
        functools.partial(_head_body, nseg=nseg),
        out_shape=jax.ShapeDtypeStruct((nseg, ncls), jnp.float32),
        interpret=_interpret,
    )(x, bcol, w1, b1.reshape(1, -1), g1.reshape(1, -1), be1.reshape(1, -1),
      w2, b2.reshape(1, -1), g2.reshape(1, -1), be2.reshape(1, -1),
      cls_w, cls_b.reshape(1, -1))


def kernel(pos, batch, conv1_params, conv2_params, mlp_params, cls_W, cls_b):
    n = pos.shape[0]
    batch = batch.astype(jnp.int32)
    bcol = batch.reshape(n, 1)
    brow = batch.reshape(1, n)
    x1 = _dyn_edge_conv(pos, batch, bcol, brow, conv1_params, 20)
    x2 = _dyn_edge_conv(x1, batch, bcol, brow, conv2_params, 10)
    return _head(x2, bcol, mlp_params, cls_W, cls_b, 8)
